# baseline (speedup 1.0000x reference)
import jax
import jax.numpy as jnp
import numpy as np
from jax.experimental import pallas as pl

_W = 512
_TWO_W = 1024
_OMEGA = 100
_N_ITERS = 8
_BATCH = 256
_NPV = 16
_SQRT_W = float(np.sqrt(_W))


def _hard_thr(c):
    a = jax.lax.bitcast_convert_type(jnp.abs(c), jnp.int32)
    rows = c.shape[0]
    ones = jnp.ones((c.shape[1], 8), jnp.bfloat16)
    omega_f = jnp.float32(_OMEGA)

    def cond(state):
        bit, _, cnt_t = state
        return jnp.logical_and(bit >= 0, jnp.any(cnt_t > omega_f))

    def body(state):
        bit, t, cnt_t = state
        cand = jnp.bitwise_or(t, jnp.left_shift(jnp.int32(1), bit))
        mask = jnp.where(a >= cand, jnp.float32(1), jnp.float32(0)).astype(jnp.bfloat16)
        cnt = jax.lax.dot_general(mask, ones, (((1,), (0,)), ((), ())),
                                  preferred_element_type=jnp.float32)[:, 0:1]
        take = cnt >= omega_f
        return (bit - 1,
                jnp.where(take, cand, t),
                jnp.where(take, cnt, cnt_t))

    state = (jnp.int32(30),
             jnp.zeros((rows, 1), jnp.int32),
             jnp.full((rows, 1), a.shape[1], jnp.float32))
    _, t, _ = jax.lax.while_loop(cond, body, state)
    return jnp.where(a >= t, c, 0.0)


def _star_kernel(x_ref, pw_ref, wd_ref, wm_ref, wa_ref, ba_ref, md_ref, z_ref):
    f32 = jnp.float32
    wd = wd_ref[...]
    x = x_ref[...]

    b_mat = jax.lax.dot_general(x, 0.5 * wd, (((1,), (1,)), ((), ())),
                                preferred_element_type=f32)
    g = jax.lax.dot_general(wd, wd, (((0,), (0,)), ((), ())),
                            preferred_element_type=f32)
    ii = jax.lax.broadcasted_iota(jnp.int32, (_TWO_W, _TWO_W), 0)
    jj = jax.lax.broadcasted_iota(jnp.int32, (_TWO_W, _TWO_W), 1)
    s_mat = jnp.where(ii == jj, f32(1.0), f32(0.0)) - 0.5 * g

    z = _hard_thr(b_mat)

    def iter_body(_, z):
        c = b_mat + jax.lax.dot_general(z, s_mat, (((1,), (1,)), ((), ())),
                                        preferred_element_type=f32)
        return _hard_thr(c)

    z = jax.lax.fori_loop(0, _N_ITERS, iter_body, z)
    z_ref[...] = z

    p = z[:, :_W] * z[:, :_W] + z[:, _W:] * z[:, _W:]
    md = jnp.sum(p, axis=0, keepdims=True)
    md_n = (md - jnp.min(md)) / (jnp.max(md) - jnp.min(md) + 1e-8)

    rows = []
    for n in range(_NPV):
        h = pw_ref[n * _BATCH:(n + 1) * _BATCH, :]
        ph = h[:, :_W] * h[:, :_W] + h[:, _W:] * h[:, _W:]
        rows.append(jnp.sum(ph, axis=0, keepdims=True))
    mdp = jnp.concatenate(rows, axis=0)
    lo = jnp.min(mdp, axis=1, keepdims=True)
    hi = jnp.max(mdp, axis=1, keepdims=True)
    mdp_n = (mdp - lo) / (hi - lo + 1e-8)

    att = jnp.sum(mdp_n * md_n, axis=1, keepdims=True)
    att = att / _SQRT_W
    e = jnp.exp(att - jnp.max(att))
    sm = e / jnp.sum(e)
    a = jnp.sum(mdp_n * sm, axis=0, keepdims=True)

    am = jax.nn.sigmoid(jax.lax.dot_general(a, wm_ref[...], (((1,), (1,)), ((), ())),
                                            preferred_element_type=f32))
    aa = jax.nn.relu(jax.lax.dot_general(a, wa_ref[...], (((1,), (1,)), ((), ())),
                                         preferred_element_type=f32) + ba_ref[...])
    mo = (md_n + aa) * am
    md_ref[...] = (mo - jnp.min(mo)) / (jnp.max(mo) - jnp.min(mo) + 1e-8)


def _star_call(x, pw, wd, wm, wa, ba, interpret=False):
    return pl.pallas_call(
        _star_kernel,
        out_shape=(
            jax.ShapeDtypeStruct((1, _W), jnp.float32),
            jax.ShapeDtypeStruct((_BATCH, _TWO_W), jnp.float32),
        ),
        interpret=interpret,
    )(x, pw, wd, wm, wa, ba)


def kernel(x, prev_windows, W_d, Wm, Wa, ba):
    pw = prev_windows.reshape(-1, prev_windows.shape[-1])
    md, z = _star_call(x, pw, W_d[0], Wm, Wa, ba.reshape(1, -1))
    return md.reshape(-1), z


# speedup vs baseline: 1.6120x; 1.6120x over previous
import jax
import jax.numpy as jnp
import numpy as np
from jax.experimental import pallas as pl

_W = 512
_TWO_W = 1024
_OMEGA = 100
_N_ITERS = 8
_BATCH = 256
_NPV = 16
_SQRT_W = float(np.sqrt(_W))


def _hard_thr(c):
    a = jax.lax.bitcast_convert_type(jnp.abs(c), jnp.int32)
    rows = c.shape[0]

    def cond(state):
        bit, _, cnt_t = state
        return jnp.logical_and(bit >= 0, jnp.any(cnt_t > _OMEGA))

    def body(state):
        bit, t, cnt_t = state
        cand = jnp.bitwise_or(t, jnp.left_shift(jnp.int32(1), bit))
        m = (a >= cand).astype(jnp.int32)
        s = m[:, 0:128]
        for k in range(1, 8):
            s = s + m[:, 128 * k:128 * (k + 1)]
        cnt = jnp.sum(s, axis=1, keepdims=True)
        take = cnt >= _OMEGA
        return (bit - 1,
                jnp.where(take, cand, t),
                jnp.where(take, cnt, cnt_t))

    state = (jnp.int32(30),
             jnp.zeros((rows, 1), jnp.int32),
             jnp.full((rows, 1), a.shape[1], jnp.int32))
    _, t, _ = jax.lax.while_loop(cond, body, state)
    return jnp.where(a >= t, c, 0.0)


def _star_kernel(x_ref, pw_ref, wd_ref, wm_ref, wa_ref, ba_ref, md_ref, z_ref):
    f32 = jnp.float32
    wd = wd_ref[...]
    x = x_ref[...]

    b_mat = jax.lax.dot_general(x, 0.5 * wd, (((1,), (1,)), ((), ())),
                                preferred_element_type=f32)
    g = jax.lax.dot_general(wd, wd, (((0,), (0,)), ((), ())),
                            preferred_element_type=f32)
    ii = jax.lax.broadcasted_iota(jnp.int32, (_TWO_W, _TWO_W), 0)
    jj = jax.lax.broadcasted_iota(jnp.int32, (_TWO_W, _TWO_W), 1)
    s_mat = jnp.where(ii == jj, f32(1.0), f32(0.0)) - 0.5 * g

    z = _hard_thr(b_mat)

    def iter_body(_, z):
        c = b_mat + jax.lax.dot_general(z, s_mat, (((1,), (1,)), ((), ())),
                                        preferred_element_type=f32)
        return _hard_thr(c)

    z = jax.lax.fori_loop(0, _N_ITERS, iter_body, z)
    z_ref[...] = z

    p = z[:, :_W] * z[:, :_W] + z[:, _W:] * z[:, _W:]
    md = jnp.sum(p, axis=0, keepdims=True)
    md_n = (md - jnp.min(md)) / (jnp.max(md) - jnp.min(md) + 1e-8)

    rows = []
    for n in range(_NPV):
        h = pw_ref[n * _BATCH:(n + 1) * _BATCH, :]
        ph = h[:, :_W] * h[:, :_W] + h[:, _W:] * h[:, _W:]
        rows.append(jnp.sum(ph, axis=0, keepdims=True))
    mdp = jnp.concatenate(rows, axis=0)
    lo = jnp.min(mdp, axis=1, keepdims=True)
    hi = jnp.max(mdp, axis=1, keepdims=True)
    mdp_n = (mdp - lo) / (hi - lo + 1e-8)

    att = jnp.sum(mdp_n * md_n, axis=1, keepdims=True)
    att = att / _SQRT_W
    e = jnp.exp(att - jnp.max(att))
    sm = e / jnp.sum(e)
    a = jnp.sum(mdp_n * sm, axis=0, keepdims=True)

    am = jax.nn.sigmoid(jax.lax.dot_general(a, wm_ref[...], (((1,), (1,)), ((), ())),
                                            preferred_element_type=f32))
    aa = jax.nn.relu(jax.lax.dot_general(a, wa_ref[...], (((1,), (1,)), ((), ())),
                                         preferred_element_type=f32) + ba_ref[...])
    mo = (md_n + aa) * am
    md_ref[...] = (mo - jnp.min(mo)) / (jnp.max(mo) - jnp.min(mo) + 1e-8)


def _star_call(x, pw, wd, wm, wa, ba, interpret=False):
    return pl.pallas_call(
        _star_kernel,
        out_shape=(
            jax.ShapeDtypeStruct((1, _W), jnp.float32),
            jax.ShapeDtypeStruct((_BATCH, _TWO_W), jnp.float32),
        ),
        interpret=interpret,
    )(x, pw, wd, wm, wa, ba)


def kernel(x, prev_windows, W_d, Wm, Wa, ba):
    pw = prev_windows.reshape(-1, prev_windows.shape[-1])
    md, z = _star_call(x, pw, W_d[0], Wm, Wa, ba.reshape(1, -1))
    return md.reshape(-1), z


# speedup vs baseline: 1.7746x; 1.1009x over previous
import jax
import jax.numpy as jnp
import numpy as np
from jax.experimental import pallas as pl

_W = 512
_TWO_W = 1024
_OMEGA = 100
_N_ITERS = 8
_BATCH = 256
_NPV = 16
_SQRT_W = float(np.sqrt(_W))


def _hard_thr(c):
    a = jax.lax.bitcast_convert_type(jnp.abs(c), jnp.int32)
    rows = c.shape[0]

    def count(cand):
        m = (a >= cand).astype(jnp.int32)
        s = m[:, 0:128]
        for k in range(1, 8):
            s = s + m[:, 128 * k:128 * (k + 1)]
        return jnp.sum(s, axis=1, keepdims=True)

    top = jnp.int32(1 << 30)
    cnt30 = count(top)
    take30 = cnt30 >= _OMEGA
    t0 = jnp.where(take30, top, 0)
    cnt_t0 = jnp.where(take30, cnt30, jnp.full((rows, 1), a.shape[1], jnp.int32))

    def cond(state):
        bit, _, cnt_t = state
        return jnp.logical_and(bit >= 0, jnp.any(cnt_t > _OMEGA))

    def body(state):
        bit, t, cnt_t = state
        b_hi = jnp.left_shift(jnp.int32(1), bit)
        b_lo = jnp.left_shift(jnp.int32(1), bit - 1)
        c1 = jnp.bitwise_or(t, b_hi)
        c12 = jnp.bitwise_or(c1, b_lo)
        c2 = jnp.bitwise_or(t, b_lo)
        n1, n12, n2 = count(c1), count(c12), count(c2)
        take1 = n1 >= _OMEGA
        cnt_lo = jnp.where(take1, n12, n2)
        take2 = cnt_lo >= _OMEGA
        t_new = jnp.where(take2,
                          jnp.where(take1, c12, c2),
                          jnp.where(take1, c1, t))
        cnt_new = jnp.where(take2, cnt_lo, jnp.where(take1, n1, cnt_t))
        return bit - 2, t_new, cnt_new

    _, t, _ = jax.lax.while_loop(cond, body, (jnp.int32(29), t0, cnt_t0))
    return jnp.where(a >= t, c, 0.0)


def _star_kernel(x_ref, pw_ref, wd_ref, wm_ref, wa_ref, ba_ref, md_ref, z_ref):
    f32 = jnp.float32
    wd = wd_ref[...]
    x = x_ref[...]

    b_mat = jax.lax.dot_general(x, 0.5 * wd, (((1,), (1,)), ((), ())),
                                preferred_element_type=f32)
    g = jax.lax.dot_general(wd, wd, (((0,), (0,)), ((), ())),
                            preferred_element_type=f32)
    ii = jax.lax.broadcasted_iota(jnp.int32, (_TWO_W, _TWO_W), 0)
    jj = jax.lax.broadcasted_iota(jnp.int32, (_TWO_W, _TWO_W), 1)
    s_mat = jnp.where(ii == jj, f32(1.0), f32(0.0)) - 0.5 * g

    z = _hard_thr(b_mat)

    def iter_body(_, z):
        c = b_mat + jax.lax.dot_general(z, s_mat, (((1,), (1,)), ((), ())),
                                        preferred_element_type=f32)
        return _hard_thr(c)

    z = jax.lax.fori_loop(0, _N_ITERS, iter_body, z)
    z_ref[...] = z

    p = z[:, :_W] * z[:, :_W] + z[:, _W:] * z[:, _W:]
    md = jnp.sum(p, axis=0, keepdims=True)
    md_n = (md - jnp.min(md)) / (jnp.max(md) - jnp.min(md) + 1e-8)

    rows = []
    for n in range(_NPV):
        h = pw_ref[n * _BATCH:(n + 1) * _BATCH, :]
        ph = h[:, :_W] * h[:, :_W] + h[:, _W:] * h[:, _W:]
        rows.append(jnp.sum(ph, axis=0, keepdims=True))
    mdp = jnp.concatenate(rows, axis=0)
    lo = jnp.min(mdp, axis=1, keepdims=True)
    hi = jnp.max(mdp, axis=1, keepdims=True)
    mdp_n = (mdp - lo) / (hi - lo + 1e-8)

    att = jnp.sum(mdp_n * md_n, axis=1, keepdims=True)
    att = att / _SQRT_W
    e = jnp.exp(att - jnp.max(att))
    sm = e / jnp.sum(e)
    a = jnp.sum(mdp_n * sm, axis=0, keepdims=True)

    am = jax.nn.sigmoid(jax.lax.dot_general(a, wm_ref[...], (((1,), (1,)), ((), ())),
                                            preferred_element_type=f32))
    aa = jax.nn.relu(jax.lax.dot_general(a, wa_ref[...], (((1,), (1,)), ((), ())),
                                         preferred_element_type=f32) + ba_ref[...])
    mo = (md_n + aa) * am
    md_ref[...] = (mo - jnp.min(mo)) / (jnp.max(mo) - jnp.min(mo) + 1e-8)


def _star_call(x, pw, wd, wm, wa, ba, interpret=False):
    return pl.pallas_call(
        _star_kernel,
        out_shape=(
            jax.ShapeDtypeStruct((1, _W), jnp.float32),
            jax.ShapeDtypeStruct((_BATCH, _TWO_W), jnp.float32),
        ),
        interpret=interpret,
    )(x, pw, wd, wm, wa, ba)


def kernel(x, prev_windows, W_d, Wm, Wa, ba):
    pw = prev_windows.reshape(-1, prev_windows.shape[-1])
    md, z = _star_call(x, pw, W_d[0], Wm, Wa, ba.reshape(1, -1))
    return md.reshape(-1), z
